# Initial kernel scaffold; baseline (speedup 1.0000x reference)
import jax
import jax.numpy as jnp
from jax.experimental import pallas as pl


def kernel(input, table):
    raise NotImplementedError("write your pallas kernel here")


# speedup vs baseline: 2.1151x; 2.1151x over previous
import functools

import jax
import jax.numpy as jnp
from jax import lax
from jax.experimental import pallas as pl
from jax.experimental.pallas import tpu as pltpu
from jax.experimental.pallas import tpu_sc as plsc

_NUM_EMBEDDINGS = 100000
_SEQ_LEN = 200
_EMB_DIM = 64
_BATCH = 4096

_NW = 32
_CH = 100
_ROWS = _BATCH * _SEQ_LEN
_ROWS_PER_W = _ROWS // _NW
_CHUNKS_PER_W = _ROWS_PER_W // _CH
_LANES = 16


def _position_embedding_host():
    even_index = jnp.arange(0, _EMB_DIM, 2, dtype=jnp.float32)
    denominator = jnp.power(10000.0, even_index / _EMB_DIM)
    positions = jnp.arange(0, _SEQ_LEN, dtype=jnp.float32).reshape(_SEQ_LEN, 1)
    even_pe = jnp.sin(positions / denominator)
    odd_pe = jnp.cos(positions / denominator)
    stacked = jnp.stack([even_pe, odd_pe], axis=2)
    return stacked.reshape(_SEQ_LEN, _EMB_DIM)


def _sc_body(table_hbm, idx_hbm, pos_hbm, out_hbm, idx_v, pos_v, rows_v, sem):
    nc = 2
    wid = lax.axis_index("s") * nc + lax.axis_index("c")
    chunk0 = wid * _CHUNKS_PER_W

    pltpu.sync_copy(idx_hbm.at[pl.ds(chunk0, _CHUNKS_PER_W)], idx_v)
    pltpu.sync_copy(pos_hbm, pos_v)

    def chunk(g, carry):
        pltpu.async_copy(table_hbm.at[idx_v.at[g]], rows_v, sem).wait()
        poff = (g % 2) * _CH

        def addrow(r, c2):
            for c in range(_EMB_DIM // _LANES):
                sl = pl.ds(c * _LANES, _LANES)
                rows_v[r, sl] = rows_v[r, sl] + pos_v[poff + r, sl]
            return c2

        lax.fori_loop(0, _CH, addrow, 0)
        pltpu.sync_copy(rows_v, out_hbm.at[chunk0 + g])
        return carry

    lax.fori_loop(0, _CHUNKS_PER_W, chunk, 0)


@jax.jit
def kernel(input, table):
    pos = _position_embedding_host()
    idx2d = input.reshape(_ROWS // _CH, _CH)

    mesh = plsc.VectorSubcoreMesh(core_axis_name="c", subcore_axis_name="s")
    out = pl.kernel(
        _sc_body,
        out_type=jax.ShapeDtypeStruct((_ROWS // _CH, _CH, _EMB_DIM), jnp.float32),
        mesh=mesh,
        scratch_types=[
            pltpu.VMEM((_CHUNKS_PER_W, _CH), jnp.int32),
            pltpu.VMEM((_SEQ_LEN, _EMB_DIM), jnp.float32),
            pltpu.VMEM((_CH, _EMB_DIM), jnp.float32),
            pltpu.SemaphoreType.DMA,
        ],
        compiler_params=pltpu.CompilerParams(use_tc_tiling_on_sc=False),
    )(table, idx2d, pos)
    return out.reshape(_BATCH, _SEQ_LEN, _EMB_DIM)


# speedup vs baseline: 3.0212x; 1.4284x over previous
import functools

import jax
import jax.numpy as jnp
from jax import lax
from jax.experimental import pallas as pl
from jax.experimental.pallas import tpu as pltpu
from jax.experimental.pallas import tpu_sc as plsc

_NUM_EMBEDDINGS = 100000
_SEQ_LEN = 200
_EMB_DIM = 64
_BATCH = 4096

_NW = 32
_CH = 100
_ROWS = _BATCH * _SEQ_LEN
_ROWS_PER_W = _ROWS // _NW
_CHUNKS_PER_W = _ROWS_PER_W // _CH


def _position_embedding_host():
    even_index = jnp.arange(0, _EMB_DIM, 2, dtype=jnp.float32)
    denominator = jnp.power(10000.0, even_index / _EMB_DIM)
    positions = jnp.arange(0, _SEQ_LEN, dtype=jnp.float32).reshape(_SEQ_LEN, 1)
    even_pe = jnp.sin(positions / denominator)
    odd_pe = jnp.cos(positions / denominator)
    stacked = jnp.stack([even_pe, odd_pe], axis=2)
    return stacked.reshape(_SEQ_LEN, _EMB_DIM)


def _sc_body(table_hbm, idx_hbm, pos_hbm, out_hbm,
             idx_v, pos_v, buf_a, buf_b, sem_a, sem_b):
    nc = 2
    wid = lax.axis_index("s") * nc + lax.axis_index("c")
    chunk0 = wid * _CHUNKS_PER_W
    last = _CHUNKS_PER_W - 1

    pltpu.sync_copy(idx_hbm.at[pl.ds(chunk0, _CHUNKS_PER_W)], idx_v)
    pltpu.sync_copy(pos_hbm, pos_v)

    def fire(g, buf, sem, poff):
        def cp(r, c):
            for cidx in range(_EMB_DIM // 16):
                sl = pl.ds(cidx * 16, 16)
                buf[r, sl] = pos_v[poff + r, sl]
            return c

        lax.fori_loop(0, _CH, cp, 0, unroll=4)
        return pltpu.async_copy(table_hbm.at[idx_v.at[g]], buf, sem, add=True)

    fire(0, buf_a, sem_a, 0)

    def body(go, carry):
        g = 2 * go
        fire(g + 1, buf_b, sem_b, _CH)
        pltpu.make_async_copy(table_hbm.at[idx_v.at[g]], buf_a, sem_a).wait()
        pltpu.sync_copy(buf_a, out_hbm.at[chunk0 + g])
        fire(jnp.minimum(g + 2, last - 1), buf_a, sem_a, 0)
        pltpu.make_async_copy(table_hbm.at[idx_v.at[g]], buf_b, sem_b).wait()
        pltpu.sync_copy(buf_b, out_hbm.at[chunk0 + g + 1])
        return carry

    lax.fori_loop(0, _CHUNKS_PER_W // 2, body, 0)
    pltpu.make_async_copy(table_hbm.at[idx_v.at[0]], buf_a, sem_a).wait()


@jax.jit
def kernel(input, table):
    pos = _position_embedding_host()
    idx2d = input.reshape(_ROWS // _CH, _CH)

    mesh = plsc.VectorSubcoreMesh(core_axis_name="c", subcore_axis_name="s")
    out = pl.kernel(
        _sc_body,
        out_type=jax.ShapeDtypeStruct((_ROWS // _CH, _CH, _EMB_DIM), jnp.float32),
        mesh=mesh,
        scratch_types=[
            pltpu.VMEM((_CHUNKS_PER_W, _CH), jnp.int32),
            pltpu.VMEM((_SEQ_LEN, _EMB_DIM), jnp.float32),
            pltpu.VMEM((_CH, _EMB_DIM), jnp.float32),
            pltpu.VMEM((_CH, _EMB_DIM), jnp.float32),
            pltpu.SemaphoreType.DMA,
            pltpu.SemaphoreType.DMA,
        ],
        compiler_params=pltpu.CompilerParams(use_tc_tiling_on_sc=False),
    )(table, idx2d, pos)
    return out.reshape(_BATCH, _SEQ_LEN, _EMB_DIM)
